# baseline (speedup 1.0000x reference)
import functools

import jax
import jax.numpy as jnp
from jax.experimental import pallas as pl
from jax.experimental.pallas import tpu as pltpu
from jax.experimental.pallas import tpu_sc as plsc

S, B, D, E = 4096, 4, 768, 64
N = S * B
TM = 512
R = N // TM
NC = 2
HN = N // NC
G2 = R // NC + E
HALF = D // 2
NH = 2 * N
GW = 128


def _mm_core(wig_ref, wit_ref, gs_ref, ge_ref, xs_ref, We_ref, be_ref,
             out_ref):
    i = pl.program_id(0)
    tile = wit_ref[i]
    prev_tile = wit_ref[jnp.maximum(i - 1, 0)]
    first = jnp.logical_or(i == 0, tile != prev_tile)

    @pl.when(first)
    def _():
        out_ref[...] = jnp.zeros_like(out_ref)

    g0 = gs_ref[i]
    g1 = ge_ref[i]

    @pl.when(g1 > g0)
    def _():
        rows = tile * TM + jax.lax.broadcasted_iota(jnp.int32, (TM, 1), 0)
        mask = jnp.logical_and(rows >= g0, rows < g1)
        a = xs_ref[...].astype(jnp.bfloat16)
        w = We_ref[0].astype(jnp.bfloat16)
        y = jax.lax.dot_general(a, w, (((1,), (1,)), ((), ())),
                                preferred_element_type=jnp.float32)
        y = y + be_ref[0]
        out_ref[...] += jnp.where(mask, y, 0.0)


def _mm_body_first(wig, wit, gs, ge, xs, We, be, out):
    _mm_core(wig, wit, gs, ge, xs, We, be, out)


def _mm_body_chain(wig, wit, gs, ge, xs, We, be, prev, out):
    del prev
    _mm_core(wig, wit, gs, ge, xs, We, be, out)


def _grouped_matmul_half(wi, xs_half, We, be, ys_prev, h):
    wi_group, wi_tile, gs, ge = (a[h] for a in wi)
    base = h * (HN // TM)
    in_specs = [
        pl.BlockSpec((TM, D), lambda i, wig, wit, s, e: (wit[i], 0)),
        pl.BlockSpec((1, D, D), lambda i, wig, wit, s, e: (wig[i], 0, 0)),
        pl.BlockSpec((1, 1, D), lambda i, wig, wit, s, e: (wig[i], 0, 0)),
    ]
    args = [wi_group, wi_tile, gs, ge, xs_half, We, be.reshape(E, 1, D)]
    if ys_prev is None:
        body, aliases = _mm_body_first, {}
    else:
        body, aliases = _mm_body_chain, {7: 0}
        in_specs.append(pl.BlockSpec(memory_space=pltpu.MemorySpace.HBM))
        args.append(ys_prev)
    grid_spec = pltpu.PrefetchScalarGridSpec(
        num_scalar_prefetch=4,
        grid=(G2,),
        in_specs=in_specs,
        out_specs=pl.BlockSpec((TM, D),
                               lambda i, wig, wit, s, e: (base + wit[i], 0)),
    )
    return pl.pallas_call(
        body,
        grid_spec=grid_spec,
        out_shape=jax.ShapeDtypeStruct((N, D), jnp.float32),
        input_output_aliases=aliases,
    )(*args)


def _sc_gather(xh, idx2):
    mesh = plsc.VectorSubcoreMesh(core_axis_name="c", subcore_axis_name="s")
    n_idx = idx2.shape[0]

    @pl.kernel(out_type=jax.ShapeDtypeStruct((n_idx, HALF), xh.dtype),
               mesh=mesh)
    def gather_kernel(x_hbm, i_hbm, o_hbm):
        def body(i_vmem, o_vmem):
            pltpu.sync_copy(x_hbm.at[i_vmem.at[0]], o_vmem)

        pltpu.emit_pipeline(
            body,
            grid=(n_idx // GW,),
            in_specs=[pl.BlockSpec((1, GW), lambda i: (0, i))],
            out_specs=[pl.BlockSpec((GW, HALF), lambda i: (i, 0))],
            core_axis_name=("c", "s"),
            dimension_semantics=(pltpu.PARALLEL,),
        )(i_hbm, o_hbm)

    return gather_kernel(xh, idx2.reshape(1, n_idx))


def _sc_scatter(ys, idx2):
    mesh = plsc.VectorSubcoreMesh(core_axis_name="c", subcore_axis_name="s")

    @pl.kernel(out_type=jax.ShapeDtypeStruct((NH, HALF), jnp.float32), mesh=mesh)
    def scatter_kernel(y_hbm, i_hbm, o_hbm):
        def body(y_vmem, i_vmem):
            pltpu.sync_copy(y_vmem, o_hbm.at[i_vmem.at[0]])

        pltpu.emit_pipeline(
            body,
            grid=(NH // GW,),
            in_specs=[
                pl.BlockSpec((GW, HALF), lambda i: (i, 0)),
                pl.BlockSpec((1, GW), lambda i: (0, i)),
            ],
            out_specs=[],
            core_axis_name=("c", "s"),
            dimension_semantics=(pltpu.PARALLEL,),
        )(y_hbm, i_hbm)

    return scatter_kernel(ys.reshape(NH, HALF), idx2.reshape(1, NH))


def _work_items_half(counts, offsets, h):
    lo, hi = h * HN, (h + 1) * HN
    cs = jnp.clip(offsets[:E], lo, hi) - lo
    ce = jnp.clip(offsets[1:], lo, hi) - lo
    first_tile = cs // TM
    last_tile = jnp.maximum(ce - 1, 0) // TM
    ntiles = jnp.where(ce > cs, last_tile - first_tile + 1, 0).astype(jnp.int32)
    wcum = jnp.concatenate(
        [jnp.zeros((1,), jnp.int32), jnp.cumsum(ntiles).astype(jnp.int32)])
    W = wcum[E]

    i = jnp.arange(G2, dtype=jnp.int32)
    g = jnp.clip(jnp.searchsorted(wcum, i, side="right") - 1, 0, E - 1)
    g = g.astype(jnp.int32)
    tile = first_tile[g] + (i - wcum[g])
    gs = cs[g]
    ge = ce[g]

    valid = i < W
    last = jnp.maximum(W - 1, 0)
    wi_tile = jnp.where(valid, tile, jnp.take(tile, last)).astype(jnp.int32)
    wi_group = jnp.where(valid, g, jnp.take(g, last)).astype(jnp.int32)
    gs = jnp.where(valid, gs, 0).astype(jnp.int32)
    ge = jnp.where(valid, ge, 0).astype(jnp.int32)
    return wi_group, wi_tile, gs, ge


def _work_items(counts):
    offsets = jnp.concatenate(
        [jnp.zeros((1,), jnp.int32), jnp.cumsum(counts).astype(jnp.int32)])
    halves = [_work_items_half(counts, offsets, h) for h in range(NC)]
    return tuple(jnp.stack([hv[k] for hv in halves]) for k in range(4))


@jax.jit
def kernel(x, gate_W, gate_b, We, be):
    gate_logits = x @ gate_W.T + gate_b
    sel = jnp.argmax(gate_logits, axis=-1).reshape(N).astype(jnp.int32)

    order = jnp.argsort(sel).astype(jnp.int32)
    idx2 = (order[:, None] * 2 + jnp.arange(2, dtype=jnp.int32)).reshape(NH)
    counts = jnp.bincount(sel, length=E)
    wi_group, wi_tile, gs, ge = _work_items(counts)

    wi = (wi_group, wi_tile, gs, ge)
    xh = x.reshape(NH, HALF)
    nhc = NH // NC
    ys = None
    for h in range(NC):
        xs_h = _sc_gather(xh, idx2[h * nhc:(h + 1) * nhc]).reshape(HN, D)
        ys = _grouped_matmul_half(wi, xs_h, We, be, ys, h)
    out_flat = _sc_scatter(ys.reshape(NH, HALF), idx2)
    return out_flat.reshape(S, B, D)


# speedup vs baseline: 1.0524x; 1.0524x over previous
import functools

import jax
import jax.numpy as jnp
from jax.experimental import pallas as pl
from jax.experimental.pallas import tpu as pltpu
from jax.experimental.pallas import tpu_sc as plsc

S, B, D, E = 4096, 4, 768, 64
N = S * B
TM = 512
R = N // TM
NC = 2
HN = N // NC
G2 = R // NC + E
HALF = D // 2
NH = 2 * N
GW = 128


def _mm_body(wig_ref, wit_ref, gs_ref, ge_ref, xs_ref, We_ref, be_ref, out_ref):
    h = pl.program_id(0)
    i = pl.program_id(1)
    tile = wit_ref[h, i]
    prev_tile = wit_ref[h, jnp.maximum(i - 1, 0)]
    first = jnp.logical_or(i == 0, tile != prev_tile)

    @pl.when(first)
    def _():
        out_ref[...] = jnp.zeros_like(out_ref)

    g0 = gs_ref[h, i]
    g1 = ge_ref[h, i]

    @pl.when(g1 > g0)
    def _():
        rows = tile * TM + jax.lax.broadcasted_iota(jnp.int32, (TM, 1), 0)
        mask = jnp.logical_and(rows >= g0, rows < g1)
        a = xs_ref[...].astype(jnp.bfloat16)
        w = We_ref[0].astype(jnp.bfloat16)
        y = jax.lax.dot_general(a, w, (((1,), (1,)), ((), ())),
                                preferred_element_type=jnp.float32)
        y = y + be_ref[0]
        out_ref[...] += jnp.where(mask, y, 0.0)


def _grouped_matmul(wi_group, wi_tile, gs, ge, xs, We, be):
    grid_spec = pltpu.PrefetchScalarGridSpec(
        num_scalar_prefetch=4,
        grid=(NC, G2),
        in_specs=[
            pl.BlockSpec((TM, D), lambda h, i, wig, wit, s, e: (wit[h, i], 0)),
            pl.BlockSpec((1, D, D),
                         lambda h, i, wig, wit, s, e: (wig[h, i], 0, 0)),
            pl.BlockSpec((1, 1, D),
                         lambda h, i, wig, wit, s, e: (wig[h, i], 0, 0)),
        ],
        out_specs=pl.BlockSpec((TM, D),
                               lambda h, i, wig, wit, s, e: (wit[h, i], 0)),
    )
    return pl.pallas_call(
        _mm_body,
        grid_spec=grid_spec,
        out_shape=jax.ShapeDtypeStruct((N, D), jnp.float32),
        compiler_params=pltpu.CompilerParams(
            dimension_semantics=("parallel", "arbitrary")),
    )(wi_group, wi_tile, gs, ge, xs, We, be.reshape(E, 1, D))


def _sc_gather(xh, idx2):
    mesh = plsc.VectorSubcoreMesh(core_axis_name="c", subcore_axis_name="s")

    @pl.kernel(out_type=jax.ShapeDtypeStruct((NH, HALF), xh.dtype), mesh=mesh)
    def gather_kernel(x_hbm, i_hbm, o_hbm):
        def body(i_vmem, o_vmem):
            pltpu.sync_copy(x_hbm.at[i_vmem.at[0]], o_vmem)

        pltpu.emit_pipeline(
            body,
            grid=(NH // GW,),
            in_specs=[pl.BlockSpec((1, GW), lambda i: (0, i))],
            out_specs=[pl.BlockSpec((GW, HALF), lambda i: (i, 0))],
            core_axis_name=("c", "s"),
            dimension_semantics=(pltpu.PARALLEL,),
        )(i_hbm, o_hbm)

    return gather_kernel(xh, idx2.reshape(1, NH))


def _sc_scatter(ys, idx2):
    mesh = plsc.VectorSubcoreMesh(core_axis_name="c", subcore_axis_name="s")

    @pl.kernel(out_type=jax.ShapeDtypeStruct((NH, HALF), jnp.float32), mesh=mesh)
    def scatter_kernel(y_hbm, i_hbm, o_hbm):
        def body(y_vmem, i_vmem):
            pltpu.sync_copy(y_vmem, o_hbm.at[i_vmem.at[0]])

        pltpu.emit_pipeline(
            body,
            grid=(NH // GW,),
            in_specs=[
                pl.BlockSpec((GW, HALF), lambda i: (i, 0)),
                pl.BlockSpec((1, GW), lambda i: (0, i)),
            ],
            out_specs=[],
            core_axis_name=("c", "s"),
            dimension_semantics=(pltpu.PARALLEL,),
        )(y_hbm, i_hbm)

    return scatter_kernel(ys.reshape(NH, HALF), idx2.reshape(1, NH))


def _work_items_half(counts, offsets, h):
    lo, hi = h * HN, (h + 1) * HN
    cs = jnp.clip(offsets[:E], lo, hi)
    ce = jnp.clip(offsets[1:], lo, hi)
    first_tile = cs // TM
    last_tile = jnp.maximum(ce - 1, 0) // TM
    ntiles = jnp.where(ce > cs, last_tile - first_tile + 1, 0).astype(jnp.int32)
    wcum = jnp.concatenate(
        [jnp.zeros((1,), jnp.int32), jnp.cumsum(ntiles).astype(jnp.int32)])
    W = wcum[E]

    i = jnp.arange(G2, dtype=jnp.int32)
    g = jnp.clip(jnp.searchsorted(wcum, i, side="right") - 1, 0, E - 1)
    g = g.astype(jnp.int32)
    tile = first_tile[g] + (i - wcum[g])
    gs = cs[g]
    ge = ce[g]

    valid = i < W
    last = jnp.maximum(W - 1, 0)
    wi_tile = jnp.where(valid, tile, jnp.take(tile, last)).astype(jnp.int32)
    wi_group = jnp.where(valid, g, jnp.take(g, last)).astype(jnp.int32)
    gs = jnp.where(valid, gs, 0).astype(jnp.int32)
    ge = jnp.where(valid, ge, 0).astype(jnp.int32)
    return wi_group, wi_tile, gs, ge


def _work_items(counts):
    offsets = jnp.concatenate(
        [jnp.zeros((1,), jnp.int32), jnp.cumsum(counts).astype(jnp.int32)])
    halves = [_work_items_half(counts, offsets, h) for h in range(NC)]
    return tuple(jnp.stack([hv[k] for hv in halves]) for k in range(4))


@jax.jit
def kernel(x, gate_W, gate_b, We, be):
    gate_logits = x @ gate_W.T + gate_b
    sel = jnp.argmax(gate_logits, axis=-1).reshape(N).astype(jnp.int32)

    order = jnp.argsort(sel, stable=False).astype(jnp.int32)
    idx2 = (order[:, None] * 2 + jnp.arange(2, dtype=jnp.int32)).reshape(NH)
    counts = jnp.bincount(sel, length=E)
    wi_group, wi_tile, gs, ge = _work_items(counts)

    xh = x.reshape(NH, HALF)
    xs = _sc_gather(xh, idx2).reshape(N, D)
    ys = _grouped_matmul(wi_group, wi_tile, gs, ge, xs, We, be)
    out_flat = _sc_scatter(ys.reshape(NH, HALF), idx2)
    return out_flat.reshape(S, B, D)


# speedup vs baseline: 1.0733x; 1.0199x over previous
import functools

import jax
import jax.numpy as jnp
from jax.experimental import pallas as pl
from jax.experimental.pallas import tpu as pltpu
from jax.experimental.pallas import tpu_sc as plsc

S, B, D, E = 4096, 4, 768, 64
N = S * B
TM = 512
R = N // TM
NC = 1
HN = N // NC
G2 = R // NC + E
HALF = D // 2
NH = 2 * N
GW = 128


def _mm_body(wig_ref, wit_ref, gs_ref, ge_ref, xs_ref, We_ref, be_ref, out_ref):
    h = pl.program_id(0)
    i = pl.program_id(1)
    tile = wit_ref[h, i]
    prev_tile = wit_ref[h, jnp.maximum(i - 1, 0)]
    first = jnp.logical_or(i == 0, tile != prev_tile)

    @pl.when(first)
    def _():
        out_ref[...] = jnp.zeros_like(out_ref)

    g0 = gs_ref[h, i]
    g1 = ge_ref[h, i]

    @pl.when(g1 > g0)
    def _():
        rows = tile * TM + jax.lax.broadcasted_iota(jnp.int32, (TM, 1), 0)
        mask = jnp.logical_and(rows >= g0, rows < g1)
        a = xs_ref[...].astype(jnp.bfloat16)
        w = We_ref[0].astype(jnp.bfloat16)
        y = jax.lax.dot_general(a, w, (((1,), (1,)), ((), ())),
                                preferred_element_type=jnp.float32)
        y = y + be_ref[0]
        out_ref[...] += jnp.where(mask, y, 0.0)


def _grouped_matmul(wi_group, wi_tile, gs, ge, xs, We, be):
    grid_spec = pltpu.PrefetchScalarGridSpec(
        num_scalar_prefetch=4,
        grid=(NC, G2),
        in_specs=[
            pl.BlockSpec((TM, D), lambda h, i, wig, wit, s, e: (wit[h, i], 0)),
            pl.BlockSpec((1, D, D),
                         lambda h, i, wig, wit, s, e: (wig[h, i], 0, 0)),
            pl.BlockSpec((1, 1, D),
                         lambda h, i, wig, wit, s, e: (wig[h, i], 0, 0)),
        ],
        out_specs=pl.BlockSpec((TM, D),
                               lambda h, i, wig, wit, s, e: (wit[h, i], 0)),
    )
    return pl.pallas_call(
        _mm_body,
        grid_spec=grid_spec,
        out_shape=jax.ShapeDtypeStruct((N, D), jnp.float32),
        compiler_params=pltpu.CompilerParams(
            dimension_semantics=("parallel", "arbitrary")),
    )(wi_group, wi_tile, gs, ge, xs, We, be.reshape(E, 1, D))


def _sc_gather(xh, idx2):
    mesh = plsc.VectorSubcoreMesh(core_axis_name="c", subcore_axis_name="s")

    @pl.kernel(out_type=jax.ShapeDtypeStruct((NH, HALF), xh.dtype), mesh=mesh)
    def gather_kernel(x_hbm, i_hbm, o_hbm):
        def body(i_vmem, o_vmem):
            pltpu.sync_copy(x_hbm.at[i_vmem.at[0]], o_vmem)

        pltpu.emit_pipeline(
            body,
            grid=(NH // GW,),
            in_specs=[pl.BlockSpec((1, GW), lambda i: (0, i))],
            out_specs=[pl.BlockSpec((GW, HALF), lambda i: (i, 0))],
            core_axis_name=("c", "s"),
            dimension_semantics=(pltpu.PARALLEL,),
        )(i_hbm, o_hbm)

    return gather_kernel(xh, idx2.reshape(1, NH))


def _sc_scatter(ys, idx2):
    mesh = plsc.VectorSubcoreMesh(core_axis_name="c", subcore_axis_name="s")

    @pl.kernel(out_type=jax.ShapeDtypeStruct((NH, HALF), jnp.float32), mesh=mesh)
    def scatter_kernel(y_hbm, i_hbm, o_hbm):
        def body(y_vmem, i_vmem):
            pltpu.sync_copy(y_vmem, o_hbm.at[i_vmem.at[0]])

        pltpu.emit_pipeline(
            body,
            grid=(NH // GW,),
            in_specs=[
                pl.BlockSpec((GW, HALF), lambda i: (i, 0)),
                pl.BlockSpec((1, GW), lambda i: (0, i)),
            ],
            out_specs=[],
            core_axis_name=("c", "s"),
            dimension_semantics=(pltpu.PARALLEL,),
        )(y_hbm, i_hbm)

    return scatter_kernel(ys.reshape(NH, HALF), idx2.reshape(1, NH))


def _work_items_half(counts, offsets, h):
    lo, hi = h * HN, (h + 1) * HN
    cs = jnp.clip(offsets[:E], lo, hi)
    ce = jnp.clip(offsets[1:], lo, hi)
    first_tile = cs // TM
    last_tile = jnp.maximum(ce - 1, 0) // TM
    ntiles = jnp.where(ce > cs, last_tile - first_tile + 1, 0).astype(jnp.int32)
    wcum = jnp.concatenate(
        [jnp.zeros((1,), jnp.int32), jnp.cumsum(ntiles).astype(jnp.int32)])
    W = wcum[E]

    i = jnp.arange(G2, dtype=jnp.int32)
    g = jnp.clip(jnp.searchsorted(wcum, i, side="right") - 1, 0, E - 1)
    g = g.astype(jnp.int32)
    tile = first_tile[g] + (i - wcum[g])
    gs = cs[g]
    ge = ce[g]

    valid = i < W
    last = jnp.maximum(W - 1, 0)
    wi_tile = jnp.where(valid, tile, jnp.take(tile, last)).astype(jnp.int32)
    wi_group = jnp.where(valid, g, jnp.take(g, last)).astype(jnp.int32)
    gs = jnp.where(valid, gs, 0).astype(jnp.int32)
    ge = jnp.where(valid, ge, 0).astype(jnp.int32)
    return wi_group, wi_tile, gs, ge


def _work_items(counts):
    offsets = jnp.concatenate(
        [jnp.zeros((1,), jnp.int32), jnp.cumsum(counts).astype(jnp.int32)])
    halves = [_work_items_half(counts, offsets, h) for h in range(NC)]
    return tuple(jnp.stack([hv[k] for hv in halves]) for k in range(4))


@jax.jit
def kernel(x, gate_W, gate_b, We, be):
    gate_logits = x @ gate_W.T + gate_b
    sel = jnp.argmax(gate_logits, axis=-1).reshape(N).astype(jnp.int32)

    order = jnp.argsort(sel, stable=False).astype(jnp.int32)
    idx2 = (order[:, None] * 2 + jnp.arange(2, dtype=jnp.int32)).reshape(NH)
    counts = jnp.bincount(sel, length=E)
    wi_group, wi_tile, gs, ge = _work_items(counts)

    xh = x.reshape(NH, HALF)
    xs = _sc_gather(xh, idx2).reshape(N, D)
    ys = _grouped_matmul(wi_group, wi_tile, gs, ge, xs, We, be)
    out_flat = _sc_scatter(ys.reshape(NH, HALF), idx2)
    return out_flat.reshape(S, B, D)
